# baseline (speedup 1.0000x reference)
import jax
import jax.numpy as jnp
from jax import lax
from jax.experimental import pallas as pl
from jax.experimental.pallas import tpu as pltpu

B = 4096
S = 200
D = 64
NSPAN = S // 2
BT = 64


def _body(x_ref, s_ref, e_ref, l_ref):
    s_ref[...] = x_ref[:, pl.dslice(0, NSPAN, 2), :]
    e_ref[...] = x_ref[:, pl.dslice(1, NSPAN, 2), :]
    l_ref[...] = jnp.full((BT, NSPAN), 2, jnp.int32)


@jax.jit
def kernel(input):
    return pl.pallas_call(
        _body,
        grid=(B // BT,),
        in_specs=[pl.BlockSpec((BT, S, D), lambda i: (i, 0, 0))],
        out_specs=(
            pl.BlockSpec((BT, NSPAN, D), lambda i: (i, 0, 0)),
            pl.BlockSpec((BT, NSPAN, D), lambda i: (i, 0, 0)),
            pl.BlockSpec((BT, NSPAN), lambda i: (i, 0)),
        ),
        out_shape=(
            jax.ShapeDtypeStruct((B, NSPAN, D), jnp.float32),
            jax.ShapeDtypeStruct((B, NSPAN, D), jnp.float32),
            jax.ShapeDtypeStruct((B, NSPAN), jnp.int32),
        ),
    )(input)


# speedup vs baseline: 1.2236x; 1.2236x over previous
import jax
import jax.numpy as jnp
from jax.experimental import pallas as pl

B = 4096
S = 200
D = 64
NSPAN = S // 2
BT = 128


def _body(x_ref, s_ref, e_ref, l_ref):
    x = x_ref[...]
    s_ref[...] = x[:, :, :D]
    e_ref[...] = x[:, :, D:]
    l_ref[...] = jnp.full((BT, NSPAN), 2, jnp.int32)


@jax.jit
def kernel(input):
    x = input.reshape(B, NSPAN, 2 * D)
    return pl.pallas_call(
        _body,
        grid=(B // BT,),
        in_specs=[pl.BlockSpec((BT, NSPAN, 2 * D), lambda i: (i, 0, 0))],
        out_specs=(
            pl.BlockSpec((BT, NSPAN, D), lambda i: (i, 0, 0)),
            pl.BlockSpec((BT, NSPAN, D), lambda i: (i, 0, 0)),
            pl.BlockSpec((BT, NSPAN), lambda i: (i, 0)),
        ),
        out_shape=(
            jax.ShapeDtypeStruct((B, NSPAN, D), jnp.float32),
            jax.ShapeDtypeStruct((B, NSPAN, D), jnp.float32),
            jax.ShapeDtypeStruct((B, NSPAN), jnp.int32),
        ),
    )(x)
